# baseline (speedup 1.0000x reference)
import functools

import jax
import jax.numpy as jnp
from jax import lax
from jax.experimental import pallas as pl
from jax.experimental.pallas import tpu as pltpu
from jax.experimental.pallas import tpu_sc as plsc

N_NODES = 10000
D_FEAT = 256
N_GRAPHS = 64

NC = 2
NS = 16
NW = NC * NS

SPAN = N_NODES // NW
NGRP = SPAN // 16
REM = SPAN - NGRP * 16
TAIL = N_NODES - SPAN * NW
IDS_PAD = 10240


def _sc_segment_sum(x, batch_i32):
  mesh = plsc.VectorSubcoreMesh(core_axis_name="c", subcore_axis_name="s")

  @functools.partial(
      pl.kernel,
      out_type=jax.ShapeDtypeStruct((NC * N_GRAPHS, D_FEAT), jnp.float32),
      mesh=mesh,
      scratch_types=[
          pltpu.VMEM((SPAN + TAIL, D_FEAT), jnp.float32),
          pltpu.VMEM((SPAN + 2 * TAIL,), jnp.int32),
          pltpu.VMEM((N_GRAPHS, D_FEAT), jnp.float32),
          pltpu.VMEM((4, D_FEAT), jnp.float32),
          pltpu.VMEM_SHARED((NS * N_GRAPHS, D_FEAT), jnp.float32),
          pltpu.SemaphoreType.DMA,
          pltpu.SemaphoreType.DMA,
      ],
  )
  def k(x_hbm, ids_hbm, sums_hbm, rows_v, idx_v, acc_v, tmp_v, part_s,
        sem, semt):
    c = lax.axis_index("c")
    s = lax.axis_index("s")
    wid = s * NC + c
    base = wid * SPAN

    pltpu.async_copy(x_hbm.at[pl.ds(base, SPAN)], rows_v.at[pl.ds(0, SPAN)], sem)
    pltpu.async_copy(ids_hbm.at[pl.ds(base, SPAN)], idx_v.at[pl.ds(0, SPAN)], sem)

    @pl.when(wid == NW - 1)
    def _():
      pltpu.async_copy(
          x_hbm.at[pl.ds(NW * SPAN, TAIL)], rows_v.at[pl.ds(SPAN, TAIL)], semt)
      pltpu.async_copy(
          ids_hbm.at[pl.ds(NW * SPAN, TAIL)], idx_v.at[pl.ds(SPAN, TAIL)], semt)

    zero = jnp.zeros((16,), jnp.float32)

    def zrow(r, carry):
      for j in range(D_FEAT // 16):
        acc_v[r, pl.ds(16 * j, 16)] = zero
      return carry

    lax.fori_loop(0, N_GRAPHS, zrow, 0)

    pltpu.make_async_copy(
        x_hbm.at[pl.ds(0, SPAN)], rows_v.at[pl.ds(0, SPAN)], sem).wait()
    pltpu.make_async_copy(
        ids_hbm.at[pl.ds(0, SPAN)], idx_v.at[pl.ds(0, SPAN)], sem).wait()

    @pl.when(wid == NW - 1)
    def _():
      pltpu.make_async_copy(
          x_hbm.at[pl.ds(0, TAIL)], rows_v.at[pl.ds(SPAN, TAIL)], semt).wait()
      pltpu.make_async_copy(
          ids_hbm.at[pl.ds(0, TAIL)], idx_v.at[pl.ds(SPAN, TAIL)], semt).wait()

    ngroups = NGRP

    def rowgroup(t, carry):
      gvec = idx_v[pl.ds(16 * t, 16)]
      g0 = gvec[0]

      @pl.when(g0 == gvec[15])
      def _():
        for j in range(D_FEAT // 16):
          sl = pl.ds(16 * j, 16)
          v = [rows_v[16 * t + l, sl] for l in range(16)]
          while len(v) > 1:
            v = [a + b for a, b in zip(v[::2], v[1::2])]
          acc_v[g0, sl] = acc_v[g0, sl] + v[0]

      @pl.when(g0 != gvec[15])
      def _():
        for l in range(16):
          g = gvec[l]
          r = 16 * t + l
          for j in range(D_FEAT // 16):
            sl = pl.ds(16 * j, 16)
            acc_v[g, sl] = acc_v[g, sl] + rows_v[r, sl]

      return carry

    lax.fori_loop(0, ngroups, rowgroup, 0)

    nrem = jnp.where(wid == NW - 1, REM + TAIL, REM)

    def rowrem(r, carry):
      gv = idx_v[pl.ds(r, 16)]
      g = gv[0]
      for j in range(D_FEAT // 16):
        sl = pl.ds(16 * j, 16)
        acc_v[g, sl] = acc_v[g, sl] + rows_v[r, sl]
      return carry

    lax.fori_loop(NGRP * 16, NGRP * 16 + nrem, rowrem, 0)

    pltpu.sync_copy(acc_v, part_s.at[pl.ds(s * N_GRAPHS, N_GRAPHS)])
    plsc.subcore_barrier()

    r0 = 4 * s

    def strip(t, carry):
      pltpu.sync_copy(part_s.at[pl.ds(t * N_GRAPHS + r0, 4)], tmp_v)
      for r in range(4):
        for j in range(D_FEAT // 16):
          sl = pl.ds(16 * j, 16)
          acc_v[r0 + r, sl] = acc_v[r0 + r, sl] + tmp_v[r, sl]
      return carry

    def other(t):
      return jnp.where(t >= s, t + 1, t)

    def strip2(t, carry):
      return strip(other(t), carry)

    lax.fori_loop(0, NS - 1, strip2, 0)
    pltpu.sync_copy(
        acc_v.at[pl.ds(r0, 4)], sums_hbm.at[pl.ds(c * N_GRAPHS + r0, 4)])

  return k(x, batch_i32)


def _tc_mlp(sums2, ids_pad, u, W1, b1, W2, b2):
  def body(sums_ref, ids_ref, u_ref, W1_ref, b1_ref, W2_ref, b2_ref, o_ref):
    sums = sums_ref[0:N_GRAPHS, :] + sums_ref[N_GRAPHS:2 * N_GRAPHS, :]
    gid = lax.broadcasted_iota(jnp.int32, (N_GRAPHS, 1), 0)
    eq = (ids_ref[...] == gid).astype(jnp.float32)
    cnt = jnp.sum(eq, axis=1, keepdims=True)
    mean = sums / jnp.maximum(cnt, 1.0)
    d_g = u_ref.shape[1]
    z = (
        jnp.dot(u_ref[...], W1_ref[0:d_g, :], preferred_element_type=jnp.float32)
        + jnp.dot(mean, W1_ref[d_g:, :], preferred_element_type=jnp.float32)
        + b1_ref[...]
    )
    h = jnp.where(z > 0, z, jnp.exp(jnp.minimum(z, 0.0)) - 1.0)
    o_ref[...] = (
        jnp.dot(h, W2_ref[...], preferred_element_type=jnp.float32) + b2_ref[...]
    )

  return pl.pallas_call(
      body,
      out_shape=jax.ShapeDtypeStruct((u.shape[0], W2.shape[1]), jnp.float32),
  )(sums2, ids_pad, u, W1, b1.reshape(1, -1), W2, b2.reshape(1, -1))


def kernel(x, edge_index, edge_attr, u, batch, W1, b1, W2, b2):
  del edge_index, edge_attr
  batch_i32 = batch.astype(jnp.int32)
  ids_pad = jnp.full((1, IDS_PAD), N_GRAPHS, jnp.int32)
  ids_pad = lax.dynamic_update_slice(ids_pad, batch_i32.reshape(1, -1), (0, 0))
  sums32 = _sc_segment_sum(x, batch_i32)
  return _tc_mlp(sums32, ids_pad, u, W1, b1, W2, b2)


# speedup vs baseline: 1.1861x; 1.1861x over previous
import functools

import jax
import jax.numpy as jnp
from jax import lax
from jax.experimental import pallas as pl
from jax.experimental.pallas import tpu as pltpu
from jax.experimental.pallas import tpu_sc as plsc

N_NODES = 10000
D_FEAT = 256
N_GRAPHS = 64

NC = 2
NS = 16
NW = NC * NS

SPAN = N_NODES // NW
NGRP = SPAN // 16
REM = SPAN - NGRP * 16
TAIL = N_NODES - SPAN * NW
IDS_PAD = 10240


def _sc_segment_sum(x, batch_i32):
  mesh = plsc.VectorSubcoreMesh(core_axis_name="c", subcore_axis_name="s")

  @functools.partial(
      pl.kernel,
      out_type=jax.ShapeDtypeStruct((NW, N_GRAPHS, D_FEAT), jnp.float32),
      mesh=mesh,
      scratch_types=[
          pltpu.VMEM((SPAN + TAIL, D_FEAT), jnp.float32),
          pltpu.VMEM((SPAN + 2 * TAIL,), jnp.int32),
          pltpu.VMEM((N_GRAPHS, D_FEAT), jnp.float32),
          pltpu.SemaphoreType.DMA,
          pltpu.SemaphoreType.DMA,
      ],
  )
  def k(x_hbm, ids_hbm, sums_hbm, rows_v, idx_v, acc_v, sem, semt):
    c = lax.axis_index("c")
    s = lax.axis_index("s")
    wid = s * NC + c
    base = wid * SPAN

    pltpu.async_copy(x_hbm.at[pl.ds(base, SPAN)], rows_v.at[pl.ds(0, SPAN)], sem)
    pltpu.async_copy(ids_hbm.at[pl.ds(base, SPAN)], idx_v.at[pl.ds(0, SPAN)], sem)

    @pl.when(wid == NW - 1)
    def _():
      pltpu.async_copy(
          x_hbm.at[pl.ds(NW * SPAN, TAIL)], rows_v.at[pl.ds(SPAN, TAIL)], semt)
      pltpu.async_copy(
          ids_hbm.at[pl.ds(NW * SPAN, TAIL)], idx_v.at[pl.ds(SPAN, TAIL)], semt)

    zero = jnp.zeros((16,), jnp.float32)

    def zrow(r, carry):
      for j in range(D_FEAT // 16):
        acc_v[r, pl.ds(16 * j, 16)] = zero
      return carry

    lax.fori_loop(0, N_GRAPHS, zrow, 0)

    pltpu.make_async_copy(
        x_hbm.at[pl.ds(0, SPAN)], rows_v.at[pl.ds(0, SPAN)], sem).wait()
    pltpu.make_async_copy(
        ids_hbm.at[pl.ds(0, SPAN)], idx_v.at[pl.ds(0, SPAN)], sem).wait()

    @pl.when(wid == NW - 1)
    def _():
      pltpu.make_async_copy(
          x_hbm.at[pl.ds(0, TAIL)], rows_v.at[pl.ds(SPAN, TAIL)], semt).wait()
      pltpu.make_async_copy(
          ids_hbm.at[pl.ds(0, TAIL)], idx_v.at[pl.ds(SPAN, TAIL)], semt).wait()

    ngroups = NGRP

    def rowgroup(t, carry):
      gvec = idx_v[pl.ds(16 * t, 16)]
      g0 = gvec[0]

      @pl.when(g0 == gvec[15])
      def _():
        for j in range(D_FEAT // 16):
          sl = pl.ds(16 * j, 16)
          v = [rows_v[16 * t + l, sl] for l in range(16)]
          while len(v) > 1:
            v = [a + b for a, b in zip(v[::2], v[1::2])]
          acc_v[g0, sl] = acc_v[g0, sl] + v[0]

      @pl.when(g0 != gvec[15])
      def _():
        for l in range(16):
          g = gvec[l]
          r = 16 * t + l
          for j in range(D_FEAT // 16):
            sl = pl.ds(16 * j, 16)
            acc_v[g, sl] = acc_v[g, sl] + rows_v[r, sl]

      return carry

    lax.fori_loop(0, ngroups, rowgroup, 0)

    nrem = jnp.where(wid == NW - 1, REM + TAIL, REM)

    def rowrem(r, carry):
      gv = idx_v[pl.ds(r, 16)]
      g = gv[0]
      for j in range(D_FEAT // 16):
        sl = pl.ds(16 * j, 16)
        acc_v[g, sl] = acc_v[g, sl] + rows_v[r, sl]
      return carry

    lax.fori_loop(NGRP * 16, NGRP * 16 + nrem, rowrem, 0)

    pltpu.sync_copy(acc_v, sums_hbm.at[wid])

  return k(x, batch_i32)


def _tc_mlp(sums2, ids_pad, u, W1, b1, W2, b2):
  def body(sums_ref, ids_ref, u_ref, W1_ref, b1_ref, W2_ref, b2_ref, o_ref):
    sums = jnp.sum(sums_ref[...], axis=0)
    gid = lax.broadcasted_iota(jnp.int32, (N_GRAPHS, 1), 0)
    eq = (ids_ref[...] == gid).astype(jnp.float32)
    cnt = jnp.sum(eq, axis=1, keepdims=True)
    mean = sums / jnp.maximum(cnt, 1.0)
    d_g = u_ref.shape[1]
    z = (
        jnp.dot(u_ref[...], W1_ref[0:d_g, :], preferred_element_type=jnp.float32)
        + jnp.dot(mean, W1_ref[d_g:, :], preferred_element_type=jnp.float32)
        + b1_ref[...]
    )
    h = jnp.where(z > 0, z, jnp.exp(jnp.minimum(z, 0.0)) - 1.0)
    o_ref[...] = (
        jnp.dot(h, W2_ref[...], preferred_element_type=jnp.float32) + b2_ref[...]
    )

  return pl.pallas_call(
      body,
      out_shape=jax.ShapeDtypeStruct((u.shape[0], W2.shape[1]), jnp.float32),
  )(sums2, ids_pad, u, W1, b1.reshape(1, -1), W2, b2.reshape(1, -1))


def kernel(x, edge_index, edge_attr, u, batch, W1, b1, W2, b2):
  del edge_index, edge_attr
  batch_i32 = batch.astype(jnp.int32)
  ids_pad = jnp.full((1, IDS_PAD), N_GRAPHS, jnp.int32)
  ids_pad = lax.dynamic_update_slice(ids_pad, batch_i32.reshape(1, -1), (0, 0))
  sums32 = _sc_segment_sum(x, batch_i32)
  return _tc_mlp(sums32, ids_pad, u, W1, b1, W2, b2)
